# baseline (device time: 68202 ns/iter reference)
import jax
import jax.numpy as jnp
from jax import lax
from jax.experimental import pallas as pl
from jax.experimental.pallas import tpu as pltpu

B, H, D, BS = 16, 16, 64, 16
NPAGES_LOCAL = 128
NKEYS = NPAGES_LOCAL * BS
NSLOTS = 128
HPAIR = 2
SCALE = D ** -0.5
NEG = -1e30


def _partials_body(bt_ref, lens_ref, q_ref, k_ref, v_ref,
                   m_ref, l_ref, o_ref, w_scr):
    g = pl.program_id(0)

    @pl.when(g == 0)
    def _():
        my_y = lax.axis_index("y")
        bt = bt_ref[...] - my_y * NPAGES_LOCAL
        iota_j = lax.broadcasted_iota(jnp.int32, (B, NSLOTS), 1)
        valid = iota_j < lens_ref[...]
        btv = jnp.where(valid, bt, -1)
        iota_p = lax.broadcasted_iota(
            jnp.int32, (B, NPAGES_LOCAL, NSLOTS), 1)
        hits = btv[:, None, :] == iota_p
        counts = jnp.sum(hits.astype(jnp.float32), axis=2)
        expand = (
            lax.broadcasted_iota(jnp.int32, (NPAGES_LOCAL, NKEYS), 0)
            == lax.broadcasted_iota(jnp.int32, (NPAGES_LOCAL, NKEYS), 1) // BS
        ).astype(jnp.float32)
        w_scr[...] = jax.lax.dot_general(
            counts, expand, (((1,), (0,)), ((), ())),
            preferred_element_type=jnp.float32)

    w = w_scr[...]
    wpos = w > 0.0
    for t in range(HPAIR):
        qh = q_ref[t].astype(jnp.bfloat16)
        kh = k_ref[:, t * D:(t + 1) * D].astype(jnp.bfloat16)
        s = jax.lax.dot_general(qh, kh, (((1,), (1,)), ((), ())),
                                preferred_element_type=jnp.float32) * SCALE
        s = jnp.where(wpos, s, NEG)
        m_h = jnp.max(s, axis=1, keepdims=True)
        e = w * jnp.exp(s - m_h)
        l_h = jnp.sum(e, axis=1, keepdims=True)
        vh = v_ref[:, t * D:(t + 1) * D].astype(jnp.bfloat16)
        o_h = jax.lax.dot_general(e.astype(jnp.bfloat16), vh,
                                  (((1,), (0,)), ((), ())),
                                  preferred_element_type=jnp.float32)
        m_ref[t] = m_h
        l_ref[t] = l_h
        o_ref[t] = o_h


def _combine_body(m_ref, l_ref, o_ref, out_ref,
                  m_rcv, l_rcv, o_rcv, send_sems, recv_sems):
    my_x = lax.axis_index("x")
    my_y = lax.axis_index("y")
    peer = (my_x, 1 - my_y)

    barrier = pltpu.get_barrier_semaphore()
    pl.semaphore_signal(barrier, inc=1, device_id=peer,
                        device_id_type=pl.DeviceIdType.MESH)
    pl.semaphore_wait(barrier, 1)

    copies = [
        pltpu.make_async_remote_copy(
            src_ref=src, dst_ref=dst,
            send_sem=send_sems.at[i], recv_sem=recv_sems.at[i],
            device_id=peer, device_id_type=pl.DeviceIdType.MESH)
        for i, (src, dst) in enumerate(
            [(m_ref, m_rcv), (l_ref, l_rcv), (o_ref, o_rcv)])
    ]
    for c in copies:
        c.start()
    for c in copies:
        c.wait()

    for h in range(H):
        m_s, m_r = m_ref[h], m_rcv[h]
        m_n = jnp.maximum(m_s, m_r)
        a_s = jnp.exp(m_s - m_n)
        a_r = jnp.exp(m_r - m_n)
        l_n = l_ref[h] * a_s + l_rcv[h] * a_r
        o_h = o_ref[h] * a_s + o_rcv[h] * a_r
        out_ref[:, h, :] = o_h / l_n


def kernel(Q, K, V, bt, lens):
    q = Q.reshape(B, H, D).transpose(1, 0, 2)
    k = K.reshape(NKEYS, H * D)
    v = V.reshape(NKEYS, H * D)
    lens2 = lens.reshape(B, 1)

    m, l, o = pl.pallas_call(
        _partials_body,
        grid=(H // HPAIR,),
        in_specs=[
            pl.BlockSpec((B, NSLOTS), lambda g: (0, 0)),
            pl.BlockSpec((B, 1), lambda g: (0, 0)),
            pl.BlockSpec((HPAIR, B, D), lambda g: (g, 0, 0)),
            pl.BlockSpec((NKEYS, HPAIR * D), lambda g: (0, g)),
            pl.BlockSpec((NKEYS, HPAIR * D), lambda g: (0, g)),
        ],
        out_shape=[
            jax.ShapeDtypeStruct((H, B, 1), jnp.float32),
            jax.ShapeDtypeStruct((H, B, 1), jnp.float32),
            jax.ShapeDtypeStruct((H, B, D), jnp.float32),
        ],
        out_specs=[
            pl.BlockSpec((HPAIR, B, 1), lambda g: (g, 0, 0)),
            pl.BlockSpec((HPAIR, B, 1), lambda g: (g, 0, 0)),
            pl.BlockSpec((HPAIR, B, D), lambda g: (g, 0, 0)),
        ],
        scratch_shapes=[pltpu.VMEM((B, NKEYS), jnp.float32)],
        compiler_params=pltpu.CompilerParams(
            dimension_semantics=("arbitrary",)),
    )(bt, lens2, q, k, v)

    out = pl.pallas_call(
        _combine_body,
        out_shape=jax.ShapeDtypeStruct((B, H, D), jnp.float32),
        in_specs=[pl.BlockSpec(memory_space=pltpu.VMEM)] * 3,
        out_specs=pl.BlockSpec(memory_space=pltpu.VMEM),
        scratch_shapes=[
            pltpu.VMEM((H, B, 1), jnp.float32),
            pltpu.VMEM((H, B, 1), jnp.float32),
            pltpu.VMEM((H, B, D), jnp.float32),
            pltpu.SemaphoreType.DMA((3,)),
            pltpu.SemaphoreType.DMA((3,)),
        ],
        compiler_params=pltpu.CompilerParams(collective_id=0),
    )(m, l, o)
    return out.reshape(B, 1, H, D)
